# Initial kernel scaffold; baseline (speedup 1.0000x reference)
import jax
import jax.numpy as jnp
from jax.experimental import pallas as pl


def kernel(x, edge_index, edge_attr, node_ids, W_msg, b_msg, W_hid, b_hid):
    raise NotImplementedError("write your pallas kernel here")


# speedup vs baseline: 1.3880x; 1.3880x over previous
import functools

import jax
import jax.numpy as jnp
from jax import lax
from jax.experimental import pallas as pl
from jax.experimental.pallas import tpu as pltpu
from jax.experimental.pallas import tpu_sc as plsc

ALPHA = 0.01
N_NODES = 10000
N_EDGES = 320000
D_FEAT = 128
D_EDGE = 16
MSG_DIM = 2 * D_FEAT + D_EDGE
MSG_PAD = 288
HALF = MSG_PAD // 2
HID_DIM = MSG_DIM + D_FEAT

_EDGES_PER_TILE = N_EDGES // 16
_CHUNK = 80
_NCHUNK = _EDGES_PER_TILE // _CHUNK
N_PAD = 10240
_ROWS_PER_TILE = N_PAD // 16
_RCHUNK = 80


def _lrelu(v):
    return jnp.maximum(v, ALPHA * v)


def _pq_body(x_ref, a_ref, b_ref, bias_ref, p_ref, q_ref):
    x = x_ref[...]
    c = pl.program_id(0)
    p_ref[0] = jnp.dot(x, a_ref[0], preferred_element_type=jnp.float32)
    q_ref[0] = (jnp.dot(x, b_ref[0], preferred_element_type=jnp.float32)
                + bias_ref[c][None, :])


def _tc_pq(x, a_r, b_r, bias_r):
    return pl.pallas_call(
        _pq_body,
        grid=(2,),
        in_specs=[
            pl.BlockSpec((N_NODES, D_FEAT), lambda c: (0, 0)),
            pl.BlockSpec((1, D_FEAT, HALF), lambda c: (c, 0, 0)),
            pl.BlockSpec((1, D_FEAT, HALF), lambda c: (c, 0, 0)),
            pl.BlockSpec((2, HALF), lambda c: (0, 0)),
        ],
        out_specs=[
            pl.BlockSpec((1, N_NODES, HALF), lambda c: (c, 0, 0)),
            pl.BlockSpec((1, N_NODES, HALF), lambda c: (c, 0, 0)),
        ],
        out_shape=[
            jax.ShapeDtypeStruct((2, N_NODES, HALF), jnp.float32),
            jax.ShapeDtypeStruct((2, N_NODES, HALF), jnp.float32),
        ],
    )(x, a_r, b_r, bias_r)


_E_BLK = 10000


def _e_body(ea_ref, c_ref, e_ref):
    e_ref[0] = jnp.dot(ea_ref[...], c_ref[0], preferred_element_type=jnp.float32)


def _tc_e(edge_attr, c_r):
    return pl.pallas_call(
        _e_body,
        grid=(2, N_EDGES // _E_BLK),
        in_specs=[
            pl.BlockSpec((_E_BLK, D_EDGE), lambda c, i: (i, 0)),
            pl.BlockSpec((1, D_EDGE, HALF), lambda c, i: (c, 0, 0)),
        ],
        out_specs=pl.BlockSpec((1, _E_BLK, HALF), lambda c, i: (c, i, 0)),
        out_shape=jax.ShapeDtypeStruct((2, N_EDGES, HALF), jnp.float32),
    )(edge_attr, c_r)


def _sc_edge_body(p_hbm, q_hbm, e_hbm, src_hbm, dst_hbm, out_hbm,
                  src_v, dst_v, dstq_v, gp_v, gq_v, t_v, accum,
                  s1, s2, s3):
    c = lax.axis_index("c")
    s = lax.axis_index("s")
    coff = c * N_NODES

    def _zrow(r, carry):
        for kk in range(HALF // 16):
            gp_v[r, pl.ds(kk * 16, 16)] = jnp.zeros((16,), jnp.float32)
        return carry
    lax.fori_loop(0, _RCHUNK, _zrow, 0)
    for k in range(_ROWS_PER_TILE // _RCHUNK):
        pltpu.sync_copy(gp_v,
                        accum.at[pl.ds(s * _ROWS_PER_TILE + k * _RCHUNK, _RCHUNK)])
    plsc.subcore_barrier()

    def _chunk(j, carry):
        base = s * _EDGES_PER_TILE + j * _CHUNK
        pltpu.sync_copy(src_hbm.at[pl.ds(base, _CHUNK)], src_v)
        pltpu.sync_copy(dst_hbm.at[pl.ds(base, _CHUNK)], dst_v)
        for kk in range(_CHUNK // 16):
            sl = pl.ds(kk * 16, 16)
            src_v[sl] = src_v[sl] + coff
            dstq_v[sl] = dst_v[sl] + coff
        cp1 = pltpu.async_copy(p_hbm.at[src_v], gp_v, s1)
        cp2 = pltpu.async_copy(q_hbm.at[dstq_v], gq_v, s2)
        cp3 = pltpu.async_copy(e_hbm.at[pl.ds(c * N_EDGES + base, _CHUNK)], t_v, s3)
        cp1.wait()
        cp2.wait()
        cp3.wait()

        def _crow(r, carry2):
            for kk in range(HALF // 16):
                sl = pl.ds(kk * 16, 16)
                v = gp_v[r, sl] + gq_v[r, sl] + t_v[r, sl]
                t_v[r, sl] = jnp.maximum(v, ALPHA * v)
            return carry2
        lax.fori_loop(0, _CHUNK, _crow, 0)

        pltpu.sync_copy(t_v, accum.at[dst_v], add=True)
        return carry
    lax.fori_loop(0, _NCHUNK, _chunk, 0)

    plsc.subcore_barrier()
    for k in range(_ROWS_PER_TILE // _RCHUNK):
        r0 = s * _ROWS_PER_TILE + k * _RCHUNK
        pltpu.sync_copy(accum.at[pl.ds(r0, _RCHUNK)], gp_v)
        pltpu.sync_copy(gp_v, out_hbm.at[pl.ds(c * N_PAD + r0, _RCHUNK)])


def _sc_edge(p_flat, q_flat, e_flat, src, dst):
    mesh = plsc.VectorSubcoreMesh(core_axis_name="c", subcore_axis_name="s")
    kern = functools.partial(
        pl.kernel,
        mesh=mesh,
        compiler_params=pltpu.CompilerParams(use_tc_tiling_on_sc=False),
        out_type=jax.ShapeDtypeStruct((2 * N_PAD, HALF), jnp.float32),
        scratch_types=[
            pltpu.VMEM((_CHUNK,), jnp.int32),
            pltpu.VMEM((_CHUNK,), jnp.int32),
            pltpu.VMEM((_CHUNK,), jnp.int32),
            pltpu.VMEM((_CHUNK, HALF), jnp.float32),
            pltpu.VMEM((_CHUNK, HALF), jnp.float32),
            pltpu.VMEM((_CHUNK, HALF), jnp.float32),
            pltpu.VMEM_SHARED((N_PAD, HALF), jnp.float32),
            pltpu.SemaphoreType.DMA,
            pltpu.SemaphoreType.DMA,
            pltpu.SemaphoreType.DMA,
        ],
    )(_sc_edge_body)
    return kern(p_flat, q_flat, e_flat, src, dst)


_H_BLK = 1000


def _hid_body(m_ref, x_ref, whm_ref, whx_ref, bias_ref, o_ref):
    h = (jnp.dot(m_ref[0], whm_ref[0], preferred_element_type=jnp.float32)
         + jnp.dot(m_ref[1], whm_ref[1], preferred_element_type=jnp.float32)
         + jnp.dot(x_ref[...], whx_ref[...], preferred_element_type=jnp.float32)
         + bias_ref[...][None, :])
    o_ref[...] = _lrelu(h)


def _tc_hid(m_split, x, whm_r, whx, b_hid):
    return pl.pallas_call(
        _hid_body,
        grid=(N_NODES // _H_BLK,),
        in_specs=[
            pl.BlockSpec((2, _H_BLK, HALF), lambda i: (0, i, 0)),
            pl.BlockSpec((_H_BLK, D_FEAT), lambda i: (i, 0)),
            pl.BlockSpec((2, HALF, HID_DIM), lambda i: (0, 0, 0)),
            pl.BlockSpec((D_FEAT, HID_DIM), lambda i: (0, 0)),
            pl.BlockSpec((HID_DIM,), lambda i: (0,)),
        ],
        out_specs=pl.BlockSpec((_H_BLK, HID_DIM), lambda i: (i, 0)),
        out_shape=jax.ShapeDtypeStruct((N_NODES, HID_DIM), jnp.float32),
    )(m_split, x, whm_r, whx, b_hid)


def kernel(x, edge_index, edge_attr, node_ids, W_msg, b_msg, W_hid, b_hid):
    wt = W_msg.T
    pad = MSG_PAD - MSG_DIM
    a_r = jnp.pad(wt[:D_FEAT], ((0, 0), (0, pad))).reshape(
        D_FEAT, 2, HALF).transpose(1, 0, 2)
    b_r = jnp.pad(wt[D_FEAT:2 * D_FEAT], ((0, 0), (0, pad))).reshape(
        D_FEAT, 2, HALF).transpose(1, 0, 2)
    c_r = jnp.pad(wt[2 * D_FEAT:], ((0, 0), (0, pad))).reshape(
        D_EDGE, 2, HALF).transpose(1, 0, 2)
    bias_r = jnp.pad(b_msg, (0, pad)).reshape(2, HALF)

    p_s, q_s = _tc_pq(x, a_r, b_r, bias_r)
    e_s = _tc_e(edge_attr, c_r)

    src = edge_index[0].astype(jnp.int32)
    dst = edge_index[1].astype(jnp.int32)
    m_flat = _sc_edge(p_s.reshape(2 * N_NODES, HALF),
                      q_s.reshape(2 * N_NODES, HALF),
                      e_s.reshape(2 * N_EDGES, HALF),
                      src, dst)
    m_split = m_flat.reshape(2, N_PAD, HALF)[:, :N_NODES]

    whm_r = jnp.pad(W_hid.T[:MSG_DIM], ((0, pad), (0, 0))).reshape(
        2, HALF, HID_DIM)
    whx = W_hid.T[MSG_DIM:]
    return _tc_hid(m_split, x, whm_r, whx, b_hid)


# speedup vs baseline: 1.4664x; 1.0565x over previous
import functools

import jax
import jax.numpy as jnp
from jax import lax
from jax.experimental import pallas as pl
from jax.experimental.pallas import tpu as pltpu
from jax.experimental.pallas import tpu_sc as plsc

ALPHA = 0.01
N_NODES = 10000
N_EDGES = 320000
D_FEAT = 128
D_EDGE = 16
MSG_DIM = 2 * D_FEAT + D_EDGE
MSG_PAD = 288
HALF = MSG_PAD // 2
HID_DIM = MSG_DIM + D_FEAT

_EDGES_PER_TILE = N_EDGES // 16
_CHUNK = 40
_NCHUNK = _EDGES_PER_TILE // _CHUNK
N_PAD = 10240
_ROWS_PER_TILE = N_PAD // 16
_RCHUNK = 40


def _lrelu(v):
    return jnp.maximum(v, ALPHA * v)


def _pq_body(x_ref, a_ref, b_ref, bias_ref, p_ref, q_ref):
    x = x_ref[...]
    c = pl.program_id(0)
    p_ref[0] = jnp.dot(x, a_ref[0], preferred_element_type=jnp.float32)
    q_ref[0] = (jnp.dot(x, b_ref[0], preferred_element_type=jnp.float32)
                + bias_ref[c][None, :])


def _tc_pq(x, a_r, b_r, bias_r):
    return pl.pallas_call(
        _pq_body,
        grid=(2,),
        in_specs=[
            pl.BlockSpec((N_NODES, D_FEAT), lambda c: (0, 0)),
            pl.BlockSpec((1, D_FEAT, HALF), lambda c: (c, 0, 0)),
            pl.BlockSpec((1, D_FEAT, HALF), lambda c: (c, 0, 0)),
            pl.BlockSpec((2, HALF), lambda c: (0, 0)),
        ],
        out_specs=[
            pl.BlockSpec((1, N_NODES, HALF), lambda c: (c, 0, 0)),
            pl.BlockSpec((1, N_NODES, HALF), lambda c: (c, 0, 0)),
        ],
        out_shape=[
            jax.ShapeDtypeStruct((2, N_NODES, HALF), jnp.float32),
            jax.ShapeDtypeStruct((2, N_NODES, HALF), jnp.float32),
        ],
    )(x, a_r, b_r, bias_r)


_E_BLK = 10000


def _e_body(ea_ref, c_ref, e_ref):
    e_ref[0] = jnp.dot(ea_ref[...], c_ref[0], preferred_element_type=jnp.float32)


def _tc_e(edge_attr, c_r):
    return pl.pallas_call(
        _e_body,
        grid=(2, N_EDGES // _E_BLK),
        in_specs=[
            pl.BlockSpec((_E_BLK, D_EDGE), lambda c, i: (i, 0)),
            pl.BlockSpec((1, D_EDGE, HALF), lambda c, i: (c, 0, 0)),
        ],
        out_specs=pl.BlockSpec((1, _E_BLK, HALF), lambda c, i: (c, i, 0)),
        out_shape=jax.ShapeDtypeStruct((2, N_EDGES, HALF), jnp.float32),
    )(edge_attr, c_r)


def _sc_edge_body(p_hbm, q_hbm, e_hbm, srcp_hbm, dstq_hbm, dst_hbm, out_hbm,
                  si_v, dq_v, di_v, gp_v, gq_v, t_v, accum,
                  sem_g0, sem_g1, sem_i0, sem_i1):
    c = lax.axis_index("c")
    s = lax.axis_index("s")
    sem_g = (sem_g0, sem_g1)
    sem_i = (sem_i0, sem_i1)

    def _zrow(r, carry):
        for kk in range(HALF // 16):
            gp_v[0, r, pl.ds(kk * 16, 16)] = jnp.zeros((16,), jnp.float32)
        return carry
    lax.fori_loop(0, _RCHUNK, _zrow, 0)
    for k in range(_ROWS_PER_TILE // _RCHUNK):
        pltpu.sync_copy(gp_v.at[0],
                        accum.at[pl.ds(s * _ROWS_PER_TILE + k * _RCHUNK, _RCHUNK)])
    plsc.subcore_barrier()

    ebase = s * _EDGES_PER_TILE

    def _idx_start(base, b):
        h1 = pltpu.async_copy(srcp_hbm.at[pl.ds(c * N_EDGES + base, _CHUNK)],
                              si_v.at[b], sem_i[b])
        h2 = pltpu.async_copy(dstq_hbm.at[pl.ds(c * N_EDGES + base, _CHUNK)],
                              dq_v.at[b], sem_i[b])
        h3 = pltpu.async_copy(dst_hbm.at[pl.ds(base, _CHUNK)],
                              di_v.at[b], sem_i[b])
        return h1, h2, h3

    def _idx_wait(b):
        pltpu.make_async_copy(srcp_hbm.at[pl.ds(0, _CHUNK)], si_v.at[b],
                              sem_i[b]).wait()
        pltpu.make_async_copy(dstq_hbm.at[pl.ds(0, _CHUNK)], dq_v.at[b],
                              sem_i[b]).wait()
        pltpu.make_async_copy(dst_hbm.at[pl.ds(0, _CHUNK)], di_v.at[b],
                              sem_i[b]).wait()

    def _gather_start(base, b):
        pltpu.async_copy(p_hbm.at[si_v.at[b]], gp_v.at[b], sem_g[b])
        pltpu.async_copy(q_hbm.at[dq_v.at[b]], gq_v.at[b], sem_g[b])
        pltpu.async_copy(e_hbm.at[pl.ds(c * N_EDGES + base, _CHUNK)],
                         t_v.at[b], sem_g[b])

    def _gather_wait(b):
        pltpu.make_async_copy(p_hbm.at[si_v.at[b]], gp_v.at[b], sem_g[b]).wait()
        pltpu.make_async_copy(q_hbm.at[dq_v.at[b]], gq_v.at[b], sem_g[b]).wait()
        pltpu.make_async_copy(e_hbm.at[pl.ds(0, _CHUNK)], t_v.at[b],
                              sem_g[b]).wait()

    _idx_start(ebase, 0)
    _idx_wait(0)
    _gather_start(ebase, 0)

    def _pair(jp, carry):
        for b in range(2):
            g = jp * 2 + b
            nb = 1 - b
            if b == 0:
                _idx_start(ebase + (g + 1) * _CHUNK, nb)
            else:
                @pl.when(jp < _NCHUNK // 2 - 1)
                def _():
                    _idx_start(ebase + (g + 1) * _CHUNK, nb)
            _gather_wait(b)

            def _crow(r, carry2):
                for kk in range(HALF // 16):
                    sl = pl.ds(kk * 16, 16)
                    v = gp_v[b, r, sl] + gq_v[b, r, sl] + t_v[b, r, sl]
                    t_v[b, r, sl] = jnp.maximum(v, ALPHA * v)
                return carry2
            lax.fori_loop(0, _CHUNK, _crow, 0)

            if b == 0:
                _idx_wait(nb)
                _gather_start(ebase + (g + 1) * _CHUNK, nb)
            else:
                @pl.when(jp < _NCHUNK // 2 - 1)
                def _():
                    _idx_wait(nb)
                    _gather_start(ebase + (g + 1) * _CHUNK, nb)

            pltpu.sync_copy(t_v.at[b], accum.at[di_v.at[b]], add=True)
        return carry
    lax.fori_loop(0, _NCHUNK // 2, _pair, 0)

    plsc.subcore_barrier()
    for k in range(_ROWS_PER_TILE // _RCHUNK):
        r0 = s * _ROWS_PER_TILE + k * _RCHUNK
        pltpu.sync_copy(accum.at[pl.ds(r0, _RCHUNK)], gp_v.at[0])
        pltpu.sync_copy(gp_v.at[0], out_hbm.at[pl.ds(c * N_PAD + r0, _RCHUNK)])


def _sc_edge(p_flat, q_flat, e_flat, srcp, dstq, dst):
    mesh = plsc.VectorSubcoreMesh(core_axis_name="c", subcore_axis_name="s")
    kern = functools.partial(
        pl.kernel,
        mesh=mesh,
        compiler_params=pltpu.CompilerParams(use_tc_tiling_on_sc=False),
        out_type=jax.ShapeDtypeStruct((2 * N_PAD, HALF), jnp.float32),
        scratch_types=[
            pltpu.VMEM((2, _CHUNK), jnp.int32),
            pltpu.VMEM((2, _CHUNK), jnp.int32),
            pltpu.VMEM((2, _CHUNK), jnp.int32),
            pltpu.VMEM((2, _CHUNK, HALF), jnp.float32),
            pltpu.VMEM((2, _CHUNK, HALF), jnp.float32),
            pltpu.VMEM((2, _CHUNK, HALF), jnp.float32),
            pltpu.VMEM_SHARED((N_PAD, HALF), jnp.float32),
            pltpu.SemaphoreType.DMA,
            pltpu.SemaphoreType.DMA,
            pltpu.SemaphoreType.DMA,
            pltpu.SemaphoreType.DMA,
        ],
    )(_sc_edge_body)
    return kern(p_flat, q_flat, e_flat, srcp, dstq, dst)


_H_BLK = 1000


def _hid_body(m_ref, x_ref, whm_ref, whx_ref, bias_ref, o_ref):
    h = (jnp.dot(m_ref[0], whm_ref[0], preferred_element_type=jnp.float32)
         + jnp.dot(m_ref[1], whm_ref[1], preferred_element_type=jnp.float32)
         + jnp.dot(x_ref[...], whx_ref[...], preferred_element_type=jnp.float32)
         + bias_ref[...][None, :])
    o_ref[...] = _lrelu(h)


def _tc_hid(m_split, x, whm_r, whx, b_hid):
    return pl.pallas_call(
        _hid_body,
        grid=(N_NODES // _H_BLK,),
        in_specs=[
            pl.BlockSpec((2, _H_BLK, HALF), lambda i: (0, i, 0)),
            pl.BlockSpec((_H_BLK, D_FEAT), lambda i: (i, 0)),
            pl.BlockSpec((2, HALF, HID_DIM), lambda i: (0, 0, 0)),
            pl.BlockSpec((D_FEAT, HID_DIM), lambda i: (0, 0)),
            pl.BlockSpec((HID_DIM,), lambda i: (0,)),
        ],
        out_specs=pl.BlockSpec((_H_BLK, HID_DIM), lambda i: (i, 0)),
        out_shape=jax.ShapeDtypeStruct((N_NODES, HID_DIM), jnp.float32),
    )(m_split, x, whm_r, whx, b_hid)


def kernel(x, edge_index, edge_attr, node_ids, W_msg, b_msg, W_hid, b_hid):
    wt = W_msg.T
    pad = MSG_PAD - MSG_DIM
    a_r = jnp.pad(wt[:D_FEAT], ((0, 0), (0, pad))).reshape(
        D_FEAT, 2, HALF).transpose(1, 0, 2)
    b_r = jnp.pad(wt[D_FEAT:2 * D_FEAT], ((0, 0), (0, pad))).reshape(
        D_FEAT, 2, HALF).transpose(1, 0, 2)
    c_r = jnp.pad(wt[2 * D_FEAT:], ((0, 0), (0, pad))).reshape(
        D_EDGE, 2, HALF).transpose(1, 0, 2)
    bias_r = jnp.pad(b_msg, (0, pad)).reshape(2, HALF)

    p_s, q_s = _tc_pq(x, a_r, b_r, bias_r)
    e_s = _tc_e(edge_attr, c_r)

    src = edge_index[0].astype(jnp.int32)
    dst = edge_index[1].astype(jnp.int32)
    srcp = jnp.concatenate([src, src + jnp.int32(N_NODES)])
    dstq = jnp.concatenate([dst, dst + jnp.int32(N_NODES)])
    m_flat = _sc_edge(p_s.reshape(2 * N_NODES, HALF),
                      q_s.reshape(2 * N_NODES, HALF),
                      e_s.reshape(2 * N_EDGES, HALF),
                      srcp, dstq, dst)
    m_split = m_flat.reshape(2, N_PAD, HALF)[:, :N_NODES]

    whm_r = jnp.pad(W_hid.T[:MSG_DIM], ((0, pad), (0, 0))).reshape(
        2, HALF, HID_DIM)
    whx = W_hid.T[MSG_DIM:]
    return _tc_hid(m_split, x, whm_r, whx, b_hid)
